# baseline (speedup 1.0000x reference)
import functools

import jax
import jax.numpy as jnp
from jax import lax
from jax.experimental import pallas as pl
from jax.experimental.pallas import tpu as pltpu
from jax.experimental.pallas import tpu_sc as plsc

VOCAB = 100000
EMBED = 128
BATCH = 16384
CTX = 6

NC = 2
NS = 16
L = 16
NW = NC * NS
BPW = BATCH // NW
CHUNK = 64
NCHUNK = BPW // CHUNK
NBUF = 2

_MESH = plsc.VectorSubcoreMesh(
    core_axis_name="c", subcore_axis_name="s", num_cores=NC, num_subcores=NS
)


@functools.partial(
    pl.kernel,
    out_type=jax.ShapeDtypeStruct((BATCH * CTX,), jnp.float32),
    mesh=_MESH,
    scratch_types=[
        pltpu.VMEM((NCHUNK, CHUNK), jnp.int32),
        [pltpu.VMEM((NCHUNK, CHUNK), jnp.int32) for _ in range(CTX)],
        [pltpu.VMEM((CHUNK, EMBED), jnp.float32) for _ in range(NBUF)],
        [[pltpu.VMEM((CHUNK, EMBED), jnp.float32) for _ in range(CTX)]
         for _ in range(NBUF)],
        [pltpu.VMEM((CHUNK * CTX,), jnp.float32) for _ in range(NBUF)],
        [pltpu.SemaphoreType.DMA for _ in range(NBUF)],
        [pltpu.SemaphoreType.DMA for _ in range(NBUF)],
        [pltpu.SemaphoreType.DMA for _ in range(6)],
    ],
    compiler_params=pltpu.CompilerParams(needs_layout_passes=False),
)
def _w2v(center_hbm, ctxt_hbm, ctable_hbm, xtable_hbm, out_hbm,
         cidx, xidx, crows, xrows, outv, sems, osems, qsems):
    wid = lax.axis_index("s") * NC + lax.axis_index("c")
    stage = [pltpu.async_copy(center_hbm.at[wid, 0], cidx.at[0], osems[0])]
    for k in range(CTX):
        stage.append(pltpu.async_copy(ctxt_hbm.at[k, wid, 0], xidx[k].at[0],
                                      osems[0]))
    for cp in stage:
        cp.wait()

    def fire(g, buf):
        cps = [pltpu.async_copy(ctable_hbm.at[cidx.at[g]], crows[buf],
                                sems[buf])]
        for k in range(CTX):
            cps.append(pltpu.async_copy(xtable_hbm.at[xidx[k].at[g]],
                                        xrows[buf][k], sems[buf]))
        return cps

    def fire_part(g, buf, lo, n, sem):
        part = pl.ds(lo, n)
        cps = [pltpu.async_copy(ctable_hbm.at[cidx.at[g, part]],
                                crows[buf].at[part], sem)]
        for k in range(CTX):
            cps.append(pltpu.async_copy(xtable_hbm.at[xidx[k].at[g, part]],
                                        xrows[buf][k].at[part], sem))
        return cps

    pend_q = [fire_part(0, 0, q * L, L, qsems[q]) for q in range(4)]
    rest = pl.ds(1, NCHUNK - 1)
    stage = [pltpu.async_copy(center_hbm.at[wid, rest], cidx.at[rest],
                              osems[1])]
    for k in range(CTX):
        stage.append(pltpu.async_copy(ctxt_hbm.at[k, wid, rest],
                                      xidx[k].at[rest], osems[1]))
    for cp in stage:
        cp.wait()
    def do_sub(buf, sub):
        lane = lax.iota(jnp.int32, L)
        rows = lane + sub * L

        def dbody(d, accs):
            dv = jnp.bitwise_and(lane + d, EMBED - 1)
            c = plsc.load_gather(crows[buf], [rows, dv])
            return tuple(
                accs[k] + c * plsc.load_gather(xrows[buf][k], [rows, dv])
                for k in range(CTX)
            )

        accs = lax.fori_loop(
            0, EMBED, dbody,
            tuple(jnp.zeros((L,), jnp.float32) for _ in range(CTX)),
        )
        m = accs[0]
        for k in range(1, CTX):
            m = jnp.maximum(m, accs[k])
        es = [jnp.exp(a - m) for a in accs]
        tot = es[0]
        for k in range(1, CTX):
            tot = tot + es[k]
        inv = 1.0 / tot
        orow = rows * CTX
        for k in range(CTX):
            plsc.store_scatter(outv[buf], [orow + k], es[k] * inv)

    def ship_out(g, buf):
        base = wid * BPW + g * CHUNK
        return pltpu.async_copy(
            outv[buf], out_hbm.at[pl.ds(base * CTX, CHUNK * CTX)], osems[buf])

    pend_out = [None] * NBUF
    pend = [fire(1, 1)]
    for q in range(4):
        for cp in pend_q[q]:
            cp.wait()
        do_sub(0, q)
    pend_out[0] = ship_out(0, 0)
    for g in range(1, NCHUNK - 1):
        buf = g % NBUF
        if g + 1 < NCHUNK - 1:
            pend_next = [fire(g + 1, (g + 1) % NBUF)]
        else:
            nb = (NCHUNK - 1) % NBUF
            half = CHUNK // 2
            pend_next = [fire_part(NCHUNK - 1, nb, 0, half, qsems[4]),
                         fire_part(NCHUNK - 1, nb, half, half, qsems[5])]
        for cps in pend:
            for cp in cps:
                cp.wait()
        pend = pend_next
        if pend_out[buf] is not None:
            pend_out[buf].wait()
        for sub in range(CHUNK // L):
            do_sub(buf, sub)
        pend_out[buf] = ship_out(g, buf)
    buf = (NCHUNK - 1) % NBUF
    if pend_out[buf] is not None:
        pend_out[buf].wait()
    for h in range(2):
        for cp in pend[h]:
            cp.wait()
        for sub in range(h * (CHUNK // L // 2), (h + 1) * (CHUNK // L // 2)):
            do_sub(buf, sub)
    pend_out[buf] = ship_out(NCHUNK - 1, buf)
    for cp in pend_out:
        if cp is not None:
            cp.wait()


def kernel(center, context, center_table, context_table):
    center_r = center.reshape(NW, NCHUNK, CHUNK)
    ctxt_r = context.T.reshape(CTX, NW, NCHUNK, CHUNK)
    out = _w2v(center_r, ctxt_r, center_table, context_table)
    return out.reshape(BATCH, CTX)


# speedup vs baseline: 1.0001x; 1.0001x over previous
import functools

import jax
import jax.numpy as jnp
from jax import lax
from jax.experimental import pallas as pl
from jax.experimental.pallas import tpu as pltpu
from jax.experimental.pallas import tpu_sc as plsc

VOCAB = 100000
EMBED = 128
BATCH = 16384
CTX = 6

NC = 2
NS = 16
L = 16
NW = NC * NS
BPW = BATCH // NW
CHUNK = 64
NCHUNK = BPW // CHUNK
NBUF = 2

_MESH = plsc.VectorSubcoreMesh(
    core_axis_name="c", subcore_axis_name="s", num_cores=NC, num_subcores=NS
)


@functools.partial(
    pl.kernel,
    out_type=jax.ShapeDtypeStruct((BATCH * CTX,), jnp.float32),
    mesh=_MESH,
    scratch_types=[
        pltpu.VMEM((NCHUNK, CHUNK), jnp.int32),
        [pltpu.VMEM((NCHUNK, CHUNK), jnp.int32) for _ in range(CTX)],
        [pltpu.VMEM((CHUNK, EMBED), jnp.float32) for _ in range(NBUF)],
        [[pltpu.VMEM((CHUNK, EMBED), jnp.float32) for _ in range(CTX)]
         for _ in range(NBUF)],
        [pltpu.VMEM((CHUNK * CTX,), jnp.float32) for _ in range(NBUF)],
        [pltpu.SemaphoreType.DMA for _ in range(NBUF)],
        [pltpu.SemaphoreType.DMA for _ in range(NBUF)],
        [pltpu.SemaphoreType.DMA for _ in range(6)],
    ],
    compiler_params=pltpu.CompilerParams(needs_layout_passes=False),
)
def _w2v(center_hbm, ctxt_hbm, ctable_hbm, xtable_hbm, out_hbm,
         cidx, xidx, crows, xrows, outv, sems, osems, qsems):
    wid = lax.axis_index("s") * NC + lax.axis_index("c")
    stage = [pltpu.async_copy(center_hbm.at[wid, 0], cidx.at[0], osems[0])]
    for k in range(CTX):
        stage.append(pltpu.async_copy(ctxt_hbm.at[k, wid, 0], xidx[k].at[0],
                                      osems[0]))
    for cp in stage:
        cp.wait()

    def fire(g, buf):
        cps = [pltpu.async_copy(ctable_hbm.at[cidx.at[g]], crows[buf],
                                sems[buf])]
        for k in range(CTX):
            cps.append(pltpu.async_copy(xtable_hbm.at[xidx[k].at[g]],
                                        xrows[buf][k], sems[buf]))
        return cps

    def fire_part(g, buf, lo, n, sem):
        part = pl.ds(lo, n)
        cps = [pltpu.async_copy(ctable_hbm.at[cidx.at[g, part]],
                                crows[buf].at[part], sem)]
        for k in range(CTX):
            cps.append(pltpu.async_copy(xtable_hbm.at[xidx[k].at[g, part]],
                                        xrows[buf][k].at[part], sem))
        return cps

    pend_q = [fire_part(0, 0, q * (CHUNK // 2), CHUNK // 2, qsems[q])
              for q in range(2)]
    rest = pl.ds(1, NCHUNK - 1)
    stage = [pltpu.async_copy(center_hbm.at[wid, rest], cidx.at[rest],
                              osems[1])]
    for k in range(CTX):
        stage.append(pltpu.async_copy(ctxt_hbm.at[k, wid, rest],
                                      xidx[k].at[rest], osems[1]))
    for cp in stage:
        cp.wait()
    def do_sub(buf, sub):
        lane = lax.iota(jnp.int32, L)
        rows = lane + sub * L

        def dbody(d, accs):
            dv = jnp.bitwise_and(lane + d, EMBED - 1)
            c = plsc.load_gather(crows[buf], [rows, dv])
            return tuple(
                accs[k] + c * plsc.load_gather(xrows[buf][k], [rows, dv])
                for k in range(CTX)
            )

        accs = lax.fori_loop(
            0, EMBED, dbody,
            tuple(jnp.zeros((L,), jnp.float32) for _ in range(CTX)),
        )
        m = accs[0]
        for k in range(1, CTX):
            m = jnp.maximum(m, accs[k])
        es = [jnp.exp(a - m) for a in accs]
        tot = es[0]
        for k in range(1, CTX):
            tot = tot + es[k]
        inv = 1.0 / tot
        orow = rows * CTX
        for k in range(CTX):
            plsc.store_scatter(outv[buf], [orow + k], es[k] * inv)

    def ship_out(g, buf):
        base = wid * BPW + g * CHUNK
        return pltpu.async_copy(
            outv[buf], out_hbm.at[pl.ds(base * CTX, CHUNK * CTX)], osems[buf])

    pend_out = [None] * NBUF
    pend = [fire(1, 1)]
    for q in range(2):
        for cp in pend_q[q]:
            cp.wait()
        for sub in range(q * (CHUNK // L // 2), (q + 1) * (CHUNK // L // 2)):
            do_sub(0, sub)
    pend_out[0] = ship_out(0, 0)
    for g in range(1, NCHUNK - 1):
        buf = g % NBUF
        if g + 1 < NCHUNK - 1:
            pend_next = [fire(g + 1, (g + 1) % NBUF)]
        else:
            nb = (NCHUNK - 1) % NBUF
            half = CHUNK // 2
            pend_next = [fire_part(NCHUNK - 1, nb, 0, half, qsems[4]),
                         fire_part(NCHUNK - 1, nb, half, half, qsems[5])]
        for cps in pend:
            for cp in cps:
                cp.wait()
        pend = pend_next
        if pend_out[buf] is not None:
            pend_out[buf].wait()
        for sub in range(CHUNK // L):
            do_sub(buf, sub)
        pend_out[buf] = ship_out(g, buf)
    buf = (NCHUNK - 1) % NBUF
    if pend_out[buf] is not None:
        pend_out[buf].wait()
    for h in range(2):
        for cp in pend[h]:
            cp.wait()
        for sub in range(h * (CHUNK // L // 2), (h + 1) * (CHUNK // L // 2)):
            do_sub(buf, sub)
    pend_out[buf] = ship_out(NCHUNK - 1, buf)
    for cp in pend_out:
        if cp is not None:
            cp.wait()


def kernel(center, context, center_table, context_table):
    center_r = center.reshape(NW, NCHUNK, CHUNK)
    ctxt_r = context.T.reshape(CTX, NW, NCHUNK, CHUNK)
    out = _w2v(center_r, ctxt_r, center_table, context_table)
    return out.reshape(BATCH, CTX)


# speedup vs baseline: 1.0120x; 1.0118x over previous
import functools

import jax
import jax.numpy as jnp
from jax import lax
from jax.experimental import pallas as pl
from jax.experimental.pallas import tpu as pltpu
from jax.experimental.pallas import tpu_sc as plsc

VOCAB = 100000
EMBED = 128
BATCH = 16384
CTX = 6

NC = 2
NS = 16
L = 16
NW = NC * NS
BPW = BATCH // NW
CHUNK = 64
NCHUNK = BPW // CHUNK
NBUF = 2

_MESH = plsc.VectorSubcoreMesh(
    core_axis_name="c", subcore_axis_name="s", num_cores=NC, num_subcores=NS
)


@functools.partial(
    pl.kernel,
    out_type=jax.ShapeDtypeStruct((BATCH * CTX,), jnp.float32),
    mesh=_MESH,
    scratch_types=[
        pltpu.VMEM((NCHUNK, CHUNK), jnp.int32),
        [pltpu.VMEM((NCHUNK, CHUNK), jnp.int32) for _ in range(CTX)],
        [pltpu.VMEM((CHUNK, EMBED), jnp.float32) for _ in range(NBUF)],
        [[pltpu.VMEM((CHUNK, EMBED), jnp.float32) for _ in range(CTX)]
         for _ in range(NBUF)],
        [pltpu.VMEM((CHUNK * CTX,), jnp.float32) for _ in range(NBUF)],
        [pltpu.SemaphoreType.DMA for _ in range(NBUF)],
        [pltpu.SemaphoreType.DMA for _ in range(NBUF)],
        [pltpu.SemaphoreType.DMA for _ in range(6)],
    ],
    compiler_params=pltpu.CompilerParams(needs_layout_passes=False),
)
def _w2v(center_hbm, ctxt_hbm, ctable_hbm, xtable_hbm, out_hbm,
         cidx, xidx, crows, xrows, outv, sems, osems, qsems):
    wid = lax.axis_index("s") * NC + lax.axis_index("c")
    stage = [pltpu.async_copy(center_hbm.at[wid, 0], cidx.at[0], osems[0])]
    for k in range(CTX):
        stage.append(pltpu.async_copy(ctxt_hbm.at[k, wid, 0], xidx[k].at[0],
                                      osems[0]))
    for cp in stage:
        cp.wait()

    def fire(g, buf):
        cps = [pltpu.async_copy(ctable_hbm.at[cidx.at[g]], crows[buf],
                                sems[buf])]
        for k in range(CTX):
            cps.append(pltpu.async_copy(xtable_hbm.at[xidx[k].at[g]],
                                        xrows[buf][k], sems[buf]))
        return cps

    def fire_part(g, buf, lo, n, sem):
        part = pl.ds(lo, n)
        cps = [pltpu.async_copy(ctable_hbm.at[cidx.at[g, part]],
                                crows[buf].at[part], sem)]
        for k in range(CTX):
            cps.append(pltpu.async_copy(xtable_hbm.at[xidx[k].at[g, part]],
                                        xrows[buf][k].at[part], sem))
        return cps

    pend_q = [fire_part(0, 0, q * (CHUNK // 2), CHUNK // 2, qsems[q])
              for q in range(2)]
    rest = pl.ds(1, NCHUNK - 1)
    stage = [pltpu.async_copy(center_hbm.at[wid, rest], cidx.at[rest],
                              osems[1])]
    for k in range(CTX):
        stage.append(pltpu.async_copy(ctxt_hbm.at[k, wid, rest],
                                      xidx[k].at[rest], osems[1]))
    for cp in stage:
        cp.wait()
    def do_sub(buf, sub):
        lane = lax.iota(jnp.int32, L)
        rows = lane + sub * L

        def dbody(d, accs):
            dv = jnp.bitwise_and(lane + d, EMBED - 1)
            c = plsc.load_gather(crows[buf], [rows, dv])
            return tuple(
                accs[k] + c * plsc.load_gather(xrows[buf][k], [rows, dv])
                for k in range(CTX)
            )

        accs = lax.fori_loop(
            0, EMBED, dbody,
            tuple(jnp.zeros((L,), jnp.float32) for _ in range(CTX)),
        )
        m = accs[0]
        for k in range(1, CTX):
            m = jnp.maximum(m, accs[k])
        es = [jnp.exp(a - m) for a in accs]
        tot = es[0]
        for k in range(1, CTX):
            tot = tot + es[k]
        inv = 1.0 / tot
        orow = rows * CTX
        for k in range(CTX):
            plsc.store_scatter(outv[buf], [orow + k], es[k] * inv)

    def ship_out(g, buf):
        base = wid * BPW + g * CHUNK
        return pltpu.async_copy(
            outv[buf], out_hbm.at[pl.ds(base * CTX, CHUNK * CTX)], osems[buf])

    pend_out = [None] * NBUF
    pend = [fire(1, 1)]
    for q in range(2):
        for cp in pend_q[q]:
            cp.wait()
        for sub in range(q * (CHUNK // L // 2), (q + 1) * (CHUNK // L // 2)):
            do_sub(0, sub)
    pend_out[0] = ship_out(0, 0)
    for g in range(1, NCHUNK):
        buf = g % NBUF
        pend_next = [fire(g + 1, (g + 1) % NBUF)] if g + 1 < NCHUNK else []
        for cps in pend:
            for cp in cps:
                cp.wait()
        pend = pend_next
        if pend_out[buf] is not None:
            pend_out[buf].wait()
        for sub in range(CHUNK // L):
            do_sub(buf, sub)
        pend_out[buf] = ship_out(g, buf)
    for cp in pend_out:
        if cp is not None:
            cp.wait()


def kernel(center, context, center_table, context_table):
    center_r = center.reshape(NW, NCHUNK, CHUNK)
    ctxt_r = context.T.reshape(CTX, NW, NCHUNK, CHUNK)
    out = _w2v(center_r, ctxt_r, center_table, context_table)
    return out.reshape(BATCH, CTX)
